# Initial kernel scaffold; baseline (speedup 1.0000x reference)
import jax
import jax.numpy as jnp
from jax.experimental import pallas as pl


def kernel(query_input, W_in, W_q, W_k, W_v, norm_query_w, norm_retrieved_w, beta, storedpatterns):
    raise NotImplementedError("write your pallas kernel here")


# speedup vs baseline: 1.2713x; 1.2713x over previous
import jax
import jax.numpy as jnp
import numpy as np
from jax.experimental import pallas as pl
from jax.experimental.pallas import tpu as pltpu

EPS = 1e-6


def _xn_body(x_ref, w_ref, xn_ref):
    x = x_ref[...]
    ms = jnp.mean(x * x, axis=-1, keepdims=True)
    xn_ref[...] = ((x * jax.lax.rsqrt(ms + EPS)) * w_ref[...]).astype(jnp.bfloat16)


def _kv_body(sp_ref, wk_ref, wv_ref, k_ref, v_ref):
    sp = sp_ref[...]
    k_ref[...] = jax.lax.dot_general(sp, wk_ref[...], (((1,), (1,)), ((), ())))
    v_ref[...] = jax.lax.dot_general(sp, wv_ref[...], (((1,), (1,)), ((), ())))


def _attn_body(beta_ref, xn_ref, w_in_ref, w_q_ref, k_ref, v_ref,
               attn_ref, counts_ref, cacc_ref, *, m, sqrt_d):
    j = pl.program_id(0)

    @pl.when(j == 0)
    def _init_counts():
        cacc_ref[...] = jnp.zeros_like(cacc_ref)

    w_in_b = w_in_ref[...].astype(jnp.bfloat16)
    proj = jax.lax.dot_general(xn_ref[...], w_in_b, (((1,), (1,)), ((), ())),
                               preferred_element_type=jnp.float32)
    q = jax.lax.dot_general(proj, w_q_ref[...], (((1,), (1,)), ((), ())))
    raw = jax.lax.dot_general(q, k_ref[...], (((1,), (1,)), ((), ())))
    beta_c = jnp.clip(beta_ref[0], 1e-2, 1e2)
    s = beta_c * raw / sqrt_d
    mx = jnp.max(s, axis=-1, keepdims=True)
    e = jnp.exp(s - mx)
    ssum = jnp.sum(e, axis=-1, keepdims=True)
    p = e / ssum
    attn_ref[...] = jax.lax.dot_general(p, v_ref[...], (((1,), (0,)), ((), ())))

    pmax = jnp.max(p, axis=-1, keepdims=True)
    miota = jax.lax.broadcasted_iota(jnp.int32, p.shape, 1)
    idx = jnp.min(jnp.where(p == pmax, miota, m), axis=-1, keepdims=True)
    hist = jnp.sum((idx == jax.lax.broadcasted_iota(jnp.int32, p.shape, 1)
                    ).astype(jnp.int32), axis=0, keepdims=True)
    hiota = jax.lax.broadcasted_iota(jnp.int32, cacc_ref.shape, 0)
    cacc_ref[...] += jnp.where(hiota == j, hist, 0)

    @pl.when(j == pl.num_programs(0) - 1)
    def _write_counts():
        counts_ref[...] = cacc_ref[...]


def _combine_body(r_ref, x_ref, w_ref, out_ref):
    r = r_ref[...]
    ms = jnp.mean(r * r, axis=-1, keepdims=True)
    rn = (r * jax.lax.rsqrt(ms + EPS)) * w_ref[...]
    out_ref[...] = x_ref[...] + rn


def kernel(query_input, W_in, W_q, W_k, W_v, norm_query_w, norm_retrieved_w,
           beta, storedpatterns):
    b, s_len, emb = query_input.shape
    h, m, d = storedpatterns.shape
    x2d = query_input.reshape(s_len, emb)
    sp_flat = storedpatterns.reshape(h * m, d)
    nq = norm_query_w.reshape(1, emb)
    nr = norm_retrieved_w.reshape(1, emb)

    n_t = 4
    t = s_len // n_t
    xn = pl.pallas_call(
        _xn_body,
        grid=(n_t,),
        in_specs=[pl.BlockSpec((t, emb), lambda i: (i, 0)),
                  pl.BlockSpec((1, emb), lambda i: (0, 0))],
        out_specs=pl.BlockSpec((t, emb), lambda i: (i, 0)),
        out_shape=jax.ShapeDtypeStruct((s_len, emb), jnp.bfloat16),
    )(x2d, nq)

    k_flat, v_flat = pl.pallas_call(
        _kv_body,
        out_shape=[jax.ShapeDtypeStruct((h * m, d), jnp.float32)] * 2,
    )(sp_flat, W_k, W_v)

    import functools
    attn, counts = pl.pallas_call(
        functools.partial(_attn_body, m=m, sqrt_d=float(np.sqrt(d))),
        grid=(h,),
        in_specs=[
            pl.BlockSpec(memory_space=pltpu.SMEM),
            pl.BlockSpec((s_len, emb), lambda j: (0, 0)),
            pl.BlockSpec((d, emb), lambda j: (j, 0)),
            pl.BlockSpec((d, d), lambda j: (0, 0)),
            pl.BlockSpec((m, d), lambda j: (j, 0)),
            pl.BlockSpec((m, d), lambda j: (j, 0)),
        ],
        out_specs=[
            pl.BlockSpec((s_len, d), lambda j: (0, j)),
            pl.BlockSpec((h, m), lambda j: (0, 0)),
        ],
        out_shape=[
            jax.ShapeDtypeStruct((s_len, emb), jnp.float32),
            jax.ShapeDtypeStruct((h, m), jnp.int32),
        ],
        scratch_shapes=[pltpu.VMEM((h, m), jnp.int32)],
    )(beta.reshape(1), xn, W_in, W_q, k_flat, v_flat)

    n_c = 8
    tc = s_len // n_c
    combined = pl.pallas_call(
        _combine_body,
        grid=(n_c,),
        in_specs=[pl.BlockSpec((tc, emb), lambda i: (i, 0)),
                  pl.BlockSpec((tc, emb), lambda i: (i, 0)),
                  pl.BlockSpec((1, emb), lambda i: (0, 0))],
        out_specs=pl.BlockSpec((tc, emb), lambda i: (i, 0)),
        out_shape=jax.ShapeDtypeStruct((s_len, emb), jnp.float32),
    )(attn, x2d, nr)

    return combined.reshape(b, s_len, emb), counts


# speedup vs baseline: 1.3335x; 1.0489x over previous
import functools

import jax
import jax.numpy as jnp
import numpy as np
from jax.experimental import pallas as pl
from jax.experimental.pallas import tpu as pltpu

EPS = 1e-6


def _xn_body(x_ref, w_ref, xn_ref):
    x = x_ref[...]
    ms = jnp.mean(x * x, axis=-1, keepdims=True)
    xn_ref[...] = ((x * jax.lax.rsqrt(ms + EPS)) * w_ref[...]).astype(jnp.bfloat16)


def _kv_body(sp_ref, wk_ref, wv_ref, k_ref, v_ref):
    sp = sp_ref[...]
    k_ref[...] = jax.lax.dot_general(
        sp, wk_ref[...], (((1,), (1,)), ((), ()))).astype(jnp.bfloat16)
    v_ref[...] = jax.lax.dot_general(
        sp, wv_ref[...], (((1,), (1,)), ((), ()))).astype(jnp.bfloat16)


def _attn_body(scale_ref, xn_ref, w_in_ref, w_q_ref, k_ref, v_ref,
               attn_ref, counts_ref, cacc_ref):
    j = pl.program_id(0)

    @pl.when(j == 0)
    def _init_counts():
        cacc_ref[...] = jnp.zeros_like(cacc_ref)

    w_in_b = w_in_ref[...].astype(jnp.bfloat16)
    proj = jax.lax.dot_general(xn_ref[...], w_in_b, (((1,), (1,)), ((), ())),
                               preferred_element_type=jnp.float32)
    w_q_b = w_q_ref[...].astype(jnp.bfloat16)
    q = jax.lax.dot_general(proj.astype(jnp.bfloat16), w_q_b,
                            (((1,), (1,)), ((), ())),
                            preferred_element_type=jnp.float32)
    raw = jax.lax.dot_general(q.astype(jnp.bfloat16), k_ref[...],
                              (((1,), (1,)), ((), ())),
                              preferred_element_type=jnp.float32)
    s = raw * scale_ref[0]
    mx = jnp.max(s, axis=-1, keepdims=True)
    e = jnp.exp(s - mx)
    ssum = jnp.sum(e, axis=-1, keepdims=True)
    unnorm = jax.lax.dot_general(e.astype(jnp.bfloat16), v_ref[...],
                                 (((1,), (0,)), ((), ())),
                                 preferred_element_type=jnp.float32)
    attn_ref[...] = unnorm / ssum

    onehot = jnp.where(e == 1.0, 1, 0)
    hist = jnp.sum(onehot, axis=0, keepdims=True)
    hiota = jax.lax.broadcasted_iota(jnp.int32, cacc_ref.shape, 0)
    cacc_ref[...] += jnp.where(hiota == j, hist, 0)

    @pl.when(j == pl.num_programs(0) - 1)
    def _write_counts():
        counts_ref[...] = cacc_ref[...]


def _combine_body(r_ref, x_ref, w_ref, out_ref):
    r = r_ref[...]
    ms = jnp.mean(r * r, axis=-1, keepdims=True)
    rn = (r * jax.lax.rsqrt(ms + EPS)) * w_ref[...]
    out_ref[...] = x_ref[...] + rn


def kernel(query_input, W_in, W_q, W_k, W_v, norm_query_w, norm_retrieved_w,
           beta, storedpatterns):
    b, s_len, emb = query_input.shape
    h, m, d = storedpatterns.shape
    x2d = query_input.reshape(s_len, emb)
    sp_flat = storedpatterns.reshape(h * m, d)
    nq = norm_query_w.reshape(1, emb)
    nr = norm_retrieved_w.reshape(1, emb)
    beta_c = jnp.clip(beta, 1e-2, 1e2)
    scale = (beta_c / np.float32(np.sqrt(d))).reshape(1)

    n_t = 4
    t = s_len // n_t
    xn = pl.pallas_call(
        _xn_body,
        grid=(n_t,),
        in_specs=[pl.BlockSpec((t, emb), lambda i: (i, 0)),
                  pl.BlockSpec((1, emb), lambda i: (0, 0))],
        out_specs=pl.BlockSpec((t, emb), lambda i: (i, 0)),
        out_shape=jax.ShapeDtypeStruct((s_len, emb), jnp.bfloat16),
    )(x2d, nq)

    k_flat, v_flat = pl.pallas_call(
        _kv_body,
        out_shape=[jax.ShapeDtypeStruct((h * m, d), jnp.bfloat16)] * 2,
    )(sp_flat, W_k, W_v)

    attn, counts = pl.pallas_call(
        _attn_body,
        grid=(h,),
        in_specs=[
            pl.BlockSpec(memory_space=pltpu.SMEM),
            pl.BlockSpec((s_len, emb), lambda j: (0, 0)),
            pl.BlockSpec((d, emb), lambda j: (j, 0)),
            pl.BlockSpec((d, d), lambda j: (0, 0)),
            pl.BlockSpec((m, d), lambda j: (j, 0)),
            pl.BlockSpec((m, d), lambda j: (j, 0)),
        ],
        out_specs=[
            pl.BlockSpec((s_len, d), lambda j: (0, j)),
            pl.BlockSpec((h, m), lambda j: (0, 0)),
        ],
        out_shape=[
            jax.ShapeDtypeStruct((s_len, emb), jnp.float32),
            jax.ShapeDtypeStruct((h, m), jnp.int32),
        ],
        scratch_shapes=[pltpu.VMEM((h, m), jnp.int32)],
    )(scale, xn, W_in, W_q, k_flat, v_flat)

    n_c = 8
    tc = s_len // n_c
    combined = pl.pallas_call(
        _combine_body,
        grid=(n_c,),
        in_specs=[pl.BlockSpec((tc, emb), lambda i: (i, 0)),
                  pl.BlockSpec((tc, emb), lambda i: (i, 0)),
                  pl.BlockSpec((1, emb), lambda i: (0, 0))],
        out_specs=pl.BlockSpec((tc, emb), lambda i: (i, 0)),
        out_shape=jax.ShapeDtypeStruct((s_len, emb), jnp.float32),
    )(attn, x2d, nr)

    return combined.reshape(b, s_len, emb), counts


# speedup vs baseline: 1.5074x; 1.1305x over previous
import jax
import jax.numpy as jnp
import numpy as np
from jax.experimental import pallas as pl
from jax.experimental.pallas import tpu as pltpu

EPS = 1e-6


def _xn_body(x_ref, w_ref, xn_ref):
    x = x_ref[...]
    ms = jnp.mean(x * x, axis=-1, keepdims=True)
    xn_ref[...] = ((x * jax.lax.rsqrt(ms + EPS)) * w_ref[...]).astype(jnp.bfloat16)


def _attn_body(scale_ref, xn_ref, w_in_ref, w_q_ref, wk_ref, wv_ref, sp_ref,
               attn_ref, counts_ref, s_scr, v_scr, cacc_ref):
    j = pl.program_id(0)
    nh = pl.num_programs(0) - 1

    @pl.when(j == 0)
    def _init_counts():
        cacc_ref[...] = jnp.zeros_like(cacc_ref)

    @pl.when(j < nh)
    def _produce():
        sp_b = sp_ref[...].astype(jnp.bfloat16)
        wk_b = wk_ref[...].astype(jnp.bfloat16)
        wv_b = wv_ref[...].astype(jnp.bfloat16)
        k = jax.lax.dot_general(sp_b, wk_b, (((1,), (1,)), ((), ())),
                                preferred_element_type=jnp.float32)
        v = jax.lax.dot_general(sp_b, wv_b, (((1,), (1,)), ((), ())),
                                preferred_element_type=jnp.float32)
        v_scr[j % 2] = v.astype(jnp.bfloat16)
        w_in_b = w_in_ref[...].astype(jnp.bfloat16)
        proj = jax.lax.dot_general(xn_ref[...], w_in_b,
                                   (((1,), (1,)), ((), ())),
                                   preferred_element_type=jnp.float32)
        w_q_b = w_q_ref[...].astype(jnp.bfloat16)
        q = jax.lax.dot_general(proj.astype(jnp.bfloat16), w_q_b,
                                (((1,), (1,)), ((), ())),
                                preferred_element_type=jnp.float32)
        raw = jax.lax.dot_general(q.astype(jnp.bfloat16), k.astype(jnp.bfloat16),
                                  (((1,), (1,)), ((), ())),
                                  preferred_element_type=jnp.float32)
        s_scr[j % 2] = raw * scale_ref[0]

    @pl.when(j > 0)
    def _consume():
        jc = j - 1
        pb = jax.lax.rem(jc, 2)
        s = s_scr[pb]
        mx = jnp.max(s, axis=-1, keepdims=True)
        e = jnp.exp(s - mx)
        ssum = jnp.sum(e, axis=-1, keepdims=True)
        unnorm = jax.lax.dot_general(e.astype(jnp.bfloat16), v_scr[pb],
                                     (((1,), (0,)), ((), ())),
                                     preferred_element_type=jnp.float32)
        attn_ref[...] = (unnorm / ssum).astype(jnp.bfloat16)

        onehot = jnp.where(e == 1.0, 1.0, 0.0).astype(jnp.bfloat16)
        ones8 = jnp.ones((8, onehot.shape[0]), jnp.bfloat16)
        hist8 = jax.lax.dot_general(ones8, onehot, (((1,), (0,)), ((), ())),
                                    preferred_element_type=jnp.float32)
        hist = hist8[0:1].astype(jnp.int32)
        hiota = jax.lax.broadcasted_iota(jnp.int32, cacc_ref.shape, 0)
        cacc_ref[...] += jnp.where(hiota == jc, hist, 0)

        @pl.when(j == nh)
        def _write_counts():
            counts_ref[...] = cacc_ref[...]


def _combine_body(r_ref, x_ref, w_ref, out_ref):
    r = r_ref[...].astype(jnp.float32)
    ms = jnp.mean(r * r, axis=-1, keepdims=True)
    rn = (r * jax.lax.rsqrt(ms + EPS)) * w_ref[...]
    out_ref[...] = x_ref[...] + rn


def kernel(query_input, W_in, W_q, W_k, W_v, norm_query_w, norm_retrieved_w,
           beta, storedpatterns):
    b, s_len, emb = query_input.shape
    h, m, d = storedpatterns.shape
    x2d = query_input.reshape(s_len, emb)
    sp_flat = storedpatterns.reshape(h * m, d)
    nq = norm_query_w.reshape(1, emb)
    nr = norm_retrieved_w.reshape(1, emb)
    beta_c = jnp.clip(beta, 1e-2, 1e2)
    scale = (beta_c / np.float32(np.sqrt(d))).reshape(1)

    n_t = 4
    t = s_len // n_t
    xn = pl.pallas_call(
        _xn_body,
        grid=(n_t,),
        in_specs=[pl.BlockSpec((t, emb), lambda i: (i, 0)),
                  pl.BlockSpec((1, emb), lambda i: (0, 0))],
        out_specs=pl.BlockSpec((t, emb), lambda i: (i, 0)),
        out_shape=jax.ShapeDtypeStruct((s_len, emb), jnp.bfloat16),
    )(x2d, nq)

    nh = h
    attn, counts = pl.pallas_call(
        _attn_body,
        grid=(nh + 1,),
        in_specs=[
            pl.BlockSpec(memory_space=pltpu.SMEM),
            pl.BlockSpec((s_len, emb), lambda j: (0, 0)),
            pl.BlockSpec((d, emb), lambda j: (jnp.minimum(j, nh - 1), 0)),
            pl.BlockSpec((d, d), lambda j: (0, 0)),
            pl.BlockSpec((d, d), lambda j: (0, 0)),
            pl.BlockSpec((d, d), lambda j: (0, 0)),
            pl.BlockSpec((m, d), lambda j: (jnp.minimum(j, nh - 1), 0)),
        ],
        out_specs=[
            pl.BlockSpec((s_len, d), lambda j: (0, jnp.maximum(j - 1, 0))),
            pl.BlockSpec((h, m), lambda j: (0, 0)),
        ],
        out_shape=[
            jax.ShapeDtypeStruct((s_len, emb), jnp.bfloat16),
            jax.ShapeDtypeStruct((h, m), jnp.int32),
        ],
        scratch_shapes=[
            pltpu.VMEM((2, s_len, m), jnp.float32),
            pltpu.VMEM((2, m, d), jnp.bfloat16),
            pltpu.VMEM((h, m), jnp.int32),
        ],
    )(scale, xn, W_in, W_q, W_k, W_v, sp_flat)

    n_c = 8
    tc = s_len // n_c
    combined = pl.pallas_call(
        _combine_body,
        grid=(n_c,),
        in_specs=[pl.BlockSpec((tc, emb), lambda i: (i, 0)),
                  pl.BlockSpec((tc, emb), lambda i: (i, 0)),
                  pl.BlockSpec((1, emb), lambda i: (0, 0))],
        out_specs=pl.BlockSpec((tc, emb), lambda i: (i, 0)),
        out_shape=jax.ShapeDtypeStruct((s_len, emb), jnp.float32),
    )(attn, x2d, nr)

    return combined.reshape(b, s_len, emb), counts
